# baseline (speedup 1.0000x reference)
import functools

import jax
import jax.numpy as jnp
from jax import lax
from jax.experimental import pallas as pl
from jax.experimental.pallas import tpu as pltpu
from jax.experimental.pallas import tpu_sc as plsc

NUM_USERS = 1000000
NUM_ITEMS = 1000
EMBED_DIM = 64
BATCH = 16384
L = 16
IDX_CHUNK = 128


def _sqrt(x):
    bits = plsc.bitcast(x, jnp.int32)
    r = plsc.bitcast(jnp.int32(0x5F3759DF) - (bits >> 1), jnp.float32)
    for _ in range(3):
        r = r * (1.5 - 0.5 * x * r * r)
    return x * r


def _log1p(t):
    s = t / (2.0 + t)
    s2 = s * s
    p = 1.0 / 7.0 + s2 * (1.0 / 9.0)
    p = 1.0 / 5.0 + s2 * p
    p = 1.0 / 3.0 + s2 * p
    return 2.0 * s * (1.0 + s2 * p)


def _tile_body(user_ref, item_ref, user_e_ref, item_e_ref, user_b_ref,
               item_b_ref, pop_ref, out_ref,
               uidx, iidx, urows, irows, ubv, ibv, wpv, outv, sem):
    info = plsc.get_sparse_core_info()
    nc = info.num_cores
    wid = lax.axis_index("s") * nc + lax.axis_index("c")
    b_per_w = BATCH // (nc * info.num_subcores)
    n_chunks = b_per_w // IDX_CHUNK
    base_chunk = wid * n_chunks

    pltpu.sync_copy(user_ref.at[pl.ds(base_chunk, n_chunks)], uidx)
    pltpu.sync_copy(item_ref.at[pl.ds(base_chunk, n_chunks)], iidx)

    copies = []
    for j in range(n_chunks):
        sl = pl.ds(j * IDX_CHUNK, IDX_CHUNK)
        copies.append(pltpu.async_copy(
            user_e_ref.at[uidx.at[j]], urows.at[sl], sem))
        copies.append(pltpu.async_copy(
            item_e_ref.at[iidx.at[j]], irows.at[sl], sem))
        copies.append(pltpu.async_copy(
            user_b_ref.at[uidx.at[j]], ubv.at[sl], sem))
        copies.append(pltpu.async_copy(
            item_b_ref.at[iidx.at[j]], ibv.at[sl], sem))
        copies.append(pltpu.async_copy(
            pop_ref.at[iidx.at[j]], wpv.at[sl], sem))
    for c in copies:
        c.wait()

    def group(g, _):
        g16 = pl.multiple_of(g * L, L)
        eids = g16 + lax.iota(jnp.int32, L)
        acc = jnp.zeros((L,), jnp.float32)
        for d in range(EMBED_DIM):
            dv = jnp.full((L,), d, jnp.int32)
            uv = plsc.load_gather(urows, [eids, dv])
            iv = plsc.load_gather(irows, [eids, dv])
            acc = acc + uv * iv
        pred = jnp.where(acc <= 0.0, jnp.exp(acc), acc + 1.0)
        p = pred * _sqrt(wpv[pl.ds(g16, L)])
        ls = jnp.minimum(p, 0.0) - _log1p(jnp.exp(-jnp.abs(p)))
        outv[pl.ds(g16, L)] = ls + ubv[pl.ds(g16, L)] + ibv[pl.ds(g16, L)]
        return 0

    lax.fori_loop(0, b_per_w // L, group, 0)
    pltpu.sync_copy(outv, out_ref.at[pl.ds(wid * b_per_w, b_per_w)])


def kernel(user, item, user_e, item_e_c, user_b, item_b, pop_item):
    info = plsc.get_sparse_core_info()
    nw = info.num_cores * info.num_subcores
    b_per_w = BATCH // nw
    n_chunks = b_per_w // IDX_CHUNK

    mesh = plsc.VectorSubcoreMesh(core_axis_name="c", subcore_axis_name="s")
    run = pl.kernel(
        _tile_body,
        mesh=mesh,
        compiler_params=pltpu.CompilerParams(needs_layout_passes=False,
                                              use_tc_tiling_on_sc=False),
        out_type=jax.ShapeDtypeStruct((BATCH,), jnp.float32),
        scratch_types=[
            pltpu.VMEM((n_chunks, IDX_CHUNK), jnp.int32),
            pltpu.VMEM((n_chunks, IDX_CHUNK), jnp.int32),
            pltpu.VMEM((b_per_w, EMBED_DIM), jnp.float32),
            pltpu.VMEM((b_per_w, EMBED_DIM), jnp.float32),
            pltpu.VMEM((b_per_w,), jnp.float32),
            pltpu.VMEM((b_per_w,), jnp.float32),
            pltpu.VMEM((b_per_w,), jnp.float32),
            pltpu.VMEM((b_per_w,), jnp.float32),
            pltpu.SemaphoreType.DMA,
        ],
    )
    return run(
        user.astype(jnp.int32).reshape(nw * n_chunks, IDX_CHUNK),
        item.astype(jnp.int32).reshape(nw * n_chunks, IDX_CHUNK),
        user_e,
        item_e_c,
        user_b.reshape(NUM_USERS),
        item_b.reshape(NUM_ITEMS),
        pop_item,
    )


# speedup vs baseline: 1.0014x; 1.0014x over previous
import jax
import jax.numpy as jnp
from jax import lax
from jax.experimental import pallas as pl
from jax.experimental.pallas import tpu as pltpu
from jax.experimental.pallas import tpu_sc as plsc

NUM_USERS = 1000000
NUM_ITEMS = 1000
EMBED_DIM = 64
BATCH = 16384
L = 16
PHASE = 128


def _sqrt(x):
    bits = plsc.bitcast(x, jnp.int32)
    r = plsc.bitcast(jnp.int32(0x5F3759DF) - (bits >> 1), jnp.float32)
    for _ in range(3):
        r = r * (1.5 - 0.5 * x * r * r)
    return x * r


def _log1p(t):
    s = t / (2.0 + t)
    s2 = s * s
    p = 1.0 / 7.0 + s2 * (1.0 / 9.0)
    p = 1.0 / 5.0 + s2 * p
    p = 1.0 / 3.0 + s2 * p
    return 2.0 * s * (1.0 + s2 * p)


def _tile_body(user_ref, item_ref, urow_ref, irow_ref, uep_ref, iep_ref,
               user_b_ref, item_b_ref, pop_ref, out_ref,
               uidx, iidx, urowi, irowi, urows, irows, ubv, ibv, wpv, outv,
               sem_a, sem_b, sem_s):
    info = plsc.get_sparse_core_info()
    nc = info.num_cores
    wid = lax.axis_index("s") * nc + lax.axis_index("c")
    b_per_w = BATCH // (nc * info.num_subcores)
    n_phases = b_per_w // PHASE
    sems = [sem_a, sem_b]

    pltpu.sync_copy(user_ref.at[wid], uidx)
    pltpu.sync_copy(item_ref.at[wid], iidx)
    pltpu.sync_copy(urow_ref.at[wid], urowi)
    pltpu.sync_copy(irow_ref.at[wid], irowi)

    small = []
    for j in range(n_phases):
        sl = pl.ds(j * PHASE, PHASE)
        small.append(pltpu.async_copy(
            user_b_ref.at[uidx.at[sl]], ubv.at[sl], sem_s))
        small.append(pltpu.async_copy(
            item_b_ref.at[iidx.at[sl]], ibv.at[sl], sem_s))
        small.append(pltpu.async_copy(
            pop_ref.at[iidx.at[sl]], wpv.at[sl], sem_s))

    def fire_rows(phase):
        b = phase % 2
        sl = pl.ds(phase * PHASE, PHASE)
        return [
            pltpu.async_copy(uep_ref.at[urowi.at[sl]], urows.at[b], sems[b]),
            pltpu.async_copy(iep_ref.at[irowi.at[sl]], irows.at[b], sems[b]),
        ]

    rows = fire_rows(0)
    for c in small:
        c.wait()

    for phase in range(n_phases):
        for c in rows:
            c.wait()
        if phase + 1 < n_phases:
            rows = fire_rows(phase + 1)
        b = phase % 2

        def group(g, _):
            g16 = pl.multiple_of(g * L, L)
            q16 = phase * PHASE + g16
            eids = g16 + lax.iota(jnp.int32, L)
            ucol = (uidx[pl.ds(q16, L)] & 1) * EMBED_DIM
            icol = (iidx[pl.ds(q16, L)] & 1) * EMBED_DIM
            acc = jnp.zeros((L,), jnp.float32)
            for d in range(EMBED_DIM):
                uv = plsc.load_gather(urows.at[b], [eids, ucol + d])
                iv = plsc.load_gather(irows.at[b], [eids, icol + d])
                acc = acc + uv * iv
            pred = jnp.where(acc <= 0.0, jnp.exp(acc), acc + 1.0)
            p = pred * _sqrt(wpv[pl.ds(q16, L)])
            ls = jnp.minimum(p, 0.0) - _log1p(jnp.exp(-jnp.abs(p)))
            outv[pl.ds(q16, L)] = (
                ls + ubv[pl.ds(q16, L)] + ibv[pl.ds(q16, L)])
            return 0

        lax.fori_loop(0, PHASE // L, group, 0)

    pltpu.sync_copy(outv, out_ref.at[pl.ds(wid * b_per_w, b_per_w)])


def kernel(user, item, user_e, item_e_c, user_b, item_b, pop_item):
    info = plsc.get_sparse_core_info()
    nw = info.num_cores * info.num_subcores
    b_per_w = BATCH // nw

    user = user.astype(jnp.int32)
    item = item.astype(jnp.int32)

    mesh = plsc.VectorSubcoreMesh(core_axis_name="c", subcore_axis_name="s")
    run = pl.kernel(
        _tile_body,
        mesh=mesh,
        compiler_params=pltpu.CompilerParams(needs_layout_passes=False,
                                             use_tc_tiling_on_sc=True),
        out_type=jax.ShapeDtypeStruct((BATCH,), jnp.float32),
        scratch_types=[
            pltpu.VMEM((b_per_w,), jnp.int32),
            pltpu.VMEM((b_per_w,), jnp.int32),
            pltpu.VMEM((b_per_w,), jnp.int32),
            pltpu.VMEM((b_per_w,), jnp.int32),
            pltpu.VMEM((2, PHASE, 2 * EMBED_DIM), jnp.float32),
            pltpu.VMEM((2, PHASE, 2 * EMBED_DIM), jnp.float32),
            pltpu.VMEM((b_per_w,), jnp.float32),
            pltpu.VMEM((b_per_w,), jnp.float32),
            pltpu.VMEM((b_per_w,), jnp.float32),
            pltpu.VMEM((b_per_w,), jnp.float32),
            pltpu.SemaphoreType.DMA,
            pltpu.SemaphoreType.DMA,
            pltpu.SemaphoreType.DMA,
        ],
    )
    return run(
        user.reshape(nw, b_per_w),
        item.reshape(nw, b_per_w),
        (user >> 1).reshape(nw, b_per_w),
        (item >> 1).reshape(nw, b_per_w),
        user_e.reshape(NUM_USERS // 2, 2 * EMBED_DIM),
        item_e_c.reshape(NUM_ITEMS // 2, 2 * EMBED_DIM),
        user_b.reshape(NUM_USERS),
        item_b.reshape(NUM_ITEMS),
        pop_item,
    )
